# baseline (speedup 1.0000x reference)
import functools

import jax
import jax.numpy as jnp
from jax import lax
from jax.experimental import pallas as pl
from jax.experimental.pallas import tpu as pltpu
from jax.experimental.pallas import tpu_sc as plsc

N, E, D, U, C = 10000, 320000, 128, 64, 40
EPS_GIN = 0.5
BN_EPS = 1e-3

NC, NS = 2, 16
NW = NC * NS
B = 128
NCHUNK = E // B
KBUF = 79
KMAIN = 76
NBUF = 4
SBUF = 2
ROW_STRIDE = 624
ROW_CHUNKS = 5


def _agg_sc(y, edges, ew):
    mesh = plsc.VectorSubcoreMesh(core_axis_name="c", subcore_axis_name="s")

    @functools.partial(
        pl.kernel,
        mesh=mesh,
        compiler_params=pltpu.CompilerParams(use_tc_tiling_on_sc=False),
        out_type=jax.ShapeDtypeStruct((N, 2 * U), jnp.float32),
        scratch_types=[
            pltpu.VMEM((KBUF * B,), jnp.int32),
            pltpu.VMEM((KBUF * B,), jnp.int32),
            pltpu.VMEM((KBUF * B,), jnp.float32),
            pltpu.VMEM((NBUF, B, U), jnp.float32),
            pltpu.VMEM((SBUF, B, U), jnp.float32),
            pltpu.VMEM_SHARED((N, U), jnp.float32),
        ]
        + [pltpu.SemaphoreType.DMA] * (NBUF + SBUF),
    )
    def agg(y_hbm, edge_hbm, ew_hbm, out_hbm,
            src_v, dst_v, ew_v, rows_v, sc_v, acc, *sems):
        gsem = sems[:NBUF]
        ssem = sems[NBUF:]
        c = lax.axis_index("c")
        s = lax.axis_index("s")
        wid = c * NS + s
        k0 = (NCHUNK * wid) // NW
        nk = (NCHUNK * (wid + 1)) // NW - k0

        e0 = k0 * B
        pltpu.sync_copy(edge_hbm.at[0, pl.ds(e0, KBUF * B)], src_v)
        pltpu.sync_copy(edge_hbm.at[1, pl.ds(e0, KBUF * B)], dst_v)
        pltpu.sync_copy(ew_hbm.at[pl.ds(e0, KBUF * B)], ew_v)

        def zrow(r, _):
            for j in range(U // 16):
                rows_v[0, r, pl.ds(j * 16, 16)] = jnp.zeros((16,), jnp.float32)
            return 0
        lax.fori_loop(0, B, zrow, 0)
        r0 = s * ROW_STRIDE
        for z in range(ROW_CHUNKS):
            pltpu.sync_copy(rows_v.at[0], acc.at[pl.ds(r0 + z * B, B)])
        plsc.subcore_barrier()

        def issue_gather(k, b):
            return pltpu.async_copy(y_hbm.at[src_v.at[pl.ds(k * B, B)]],
                                    rows_v.at[b], gsem[b])

        def body(k, b, sb):
            pltpu.make_async_copy(y_hbm.at[src_v.at[pl.ds(k * B, B)]],
                                  rows_v.at[b], gsem[b]).wait()

            @pl.when(k >= SBUF)
            def _():
                pltpu.make_async_copy(sc_v.at[sb],
                                      acc.at[dst_v.at[pl.ds((k - SBUF) * B, B)]],
                                      ssem[sb]).wait()

            @plsc.parallel_loop(0, B // 16, unroll=2)
            def scale(i):
                wv = ew_v[pl.ds(k * B + i * 16, 16)]
                for g in range(4):
                    es = [i * 16 + g * 4 + l for l in range(4)]
                    vals = [rows_v[b, e, pl.ds(j * 16, 16)]
                            for e in es for j in range(U // 16)]
                    outs = [vals[n] * wv[g * 4 + n // 4]
                            for n in range(len(vals))]
                    for n, e in enumerate(es):
                        for j in range(U // 16):
                            sc_v[sb, e, pl.ds(j * 16, 16)] = outs[n * 4 + j]

            pltpu.async_copy(sc_v.at[sb], acc.at[dst_v.at[pl.ds(k * B, B)]],
                             ssem[sb], add=True)

            @pl.when(k + NBUF < nk)
            def _():
                issue_gather(k + NBUF, b)

        for b0 in range(NBUF):
            issue_gather(b0, b0)

        def quad(k4, _):
            for b in range(NBUF):
                body(k4 * NBUF + b, b, b % SBUF)
            return 0
        lax.fori_loop(0, KMAIN // NBUF, quad, 0)

        body(jnp.int32(KMAIN), 0, 0)
        body(jnp.int32(KMAIN + 1), 1, 1)

        @pl.when(nk == KBUF)
        def _():
            body(jnp.int32(KMAIN + 2), 2, 0)

        for sb in range(SBUF):
            pltpu.make_async_copy(sc_v.at[sb], acc.at[dst_v.at[pl.ds(0, B)]],
                                  ssem[sb]).wait()
        plsc.subcore_barrier()

        for z in range(ROW_CHUNKS):
            pltpu.sync_copy(acc.at[pl.ds(r0 + z * B, B)], rows_v.at[0])
            pltpu.sync_copy(rows_v.at[0],
                            out_hbm.at[pl.ds(r0 + z * B, B), pl.ds(c * U, U)])

    return agg(y, edges, ew)


_RB = 1000


def _mm1_body(x_ref, w_ref, o_ref):
    o_ref[...] = jnp.dot(x_ref[...], w_ref[...], preferred_element_type=jnp.float32)


def _mlp1_body(y_ref, p_ref, b1_ref, w2_ref, b2_ref, g1_ref, be1_ref,
               mm1_ref, mv1_ref, w3_ref, o_ref):
    p = p_ref[...]
    a = (1.0 + EPS_GIN) * y_ref[...] + p[:, :U] + p[:, U:] + b1_ref[...]
    h = jnp.maximum(a, 0.0)
    h = jnp.dot(h, w2_ref[...], preferred_element_type=jnp.float32) + b2_ref[...]
    scale = g1_ref[...] * lax.rsqrt(mv1_ref[...] + BN_EPS)
    h = (h - mm1_ref[...]) * scale + be1_ref[...]
    h = jnp.maximum(h, 0.0)
    o_ref[...] = jnp.dot(h, w3_ref[...], preferred_element_type=jnp.float32)


def _mlp2_body(y_ref, p_ref, b3_ref, w4_ref, b4_ref, g2_ref, be2_ref,
               mm2_ref, mv2_ref, o_ref):
    p = p_ref[...]
    a = (1.0 + EPS_GIN) * y_ref[...] + p[:, :U] + p[:, U:] + b3_ref[...]
    h = jnp.maximum(a, 0.0)
    o = jnp.dot(h, w4_ref[...], preferred_element_type=jnp.float32) + b4_ref[...]
    scale = g2_ref[...] * lax.rsqrt(mv2_ref[...] + BN_EPS)
    o_ref[...] = (o - mm2_ref[...]) * scale + be2_ref[...]


def _row(v):
    return v.reshape(1, -1)


def kernel(x, edge_index, edge_weight, W1, b1, W2, b2, g1, be1, mm1, mv1,
           W3, b3, W4, b4, g2, be2, mm2, mv2):
    edges = edge_index
    ew_p = edge_weight

    grid = (N // _RB,)

    y1 = pl.pallas_call(
        _mm1_body,
        grid=grid,
        in_specs=[pl.BlockSpec((_RB, D), lambda i: (i, 0)),
                  pl.BlockSpec((D, U), lambda i: (0, 0))],
        out_specs=pl.BlockSpec((_RB, U), lambda i: (i, 0)),
        out_shape=jax.ShapeDtypeStruct((N, U), jnp.float32),
    )(x, W1)

    p1 = _agg_sc(y1, edges, ew_p)

    y2 = pl.pallas_call(
        _mlp1_body,
        grid=grid,
        in_specs=[pl.BlockSpec((_RB, U), lambda i: (i, 0)),
                  pl.BlockSpec((_RB, 2 * U), lambda i: (i, 0)),
                  pl.BlockSpec((1, U), lambda i: (0, 0)),
                  pl.BlockSpec((U, U), lambda i: (0, 0)),
                  pl.BlockSpec((1, U), lambda i: (0, 0)),
                  pl.BlockSpec((1, U), lambda i: (0, 0)),
                  pl.BlockSpec((1, U), lambda i: (0, 0)),
                  pl.BlockSpec((1, U), lambda i: (0, 0)),
                  pl.BlockSpec((1, U), lambda i: (0, 0)),
                  pl.BlockSpec((U, U), lambda i: (0, 0))],
        out_specs=pl.BlockSpec((_RB, U), lambda i: (i, 0)),
        out_shape=jax.ShapeDtypeStruct((N, U), jnp.float32),
    )(y1, p1, _row(b1), W2, _row(b2), _row(g1), _row(be1), _row(mm1),
      _row(mv1), W3)

    p2 = _agg_sc(y2, edges, ew_p)

    out = pl.pallas_call(
        _mlp2_body,
        grid=grid,
        in_specs=[pl.BlockSpec((_RB, U), lambda i: (i, 0)),
                  pl.BlockSpec((_RB, 2 * U), lambda i: (i, 0)),
                  pl.BlockSpec((1, U), lambda i: (0, 0)),
                  pl.BlockSpec((U, C), lambda i: (0, 0)),
                  pl.BlockSpec((1, C), lambda i: (0, 0)),
                  pl.BlockSpec((1, C), lambda i: (0, 0)),
                  pl.BlockSpec((1, C), lambda i: (0, 0)),
                  pl.BlockSpec((1, C), lambda i: (0, 0)),
                  pl.BlockSpec((1, C), lambda i: (0, 0))],
        out_specs=pl.BlockSpec((_RB, C), lambda i: (i, 0)),
        out_shape=jax.ShapeDtypeStruct((N, C), jnp.float32),
    )(y2, p2, _row(b3), W4, _row(b4), _row(g2), _row(be2), _row(mm2),
      _row(mv2))

    return out


# speedup vs baseline: 1.0261x; 1.0261x over previous
import functools

import jax
import jax.numpy as jnp
from jax import lax
from jax.experimental import pallas as pl
from jax.experimental.pallas import tpu as pltpu
from jax.experimental.pallas import tpu_sc as plsc

N, E, D, U, C = 10000, 320000, 128, 64, 40
EPS_GIN = 0.5
BN_EPS = 1e-3

NC, NS = 2, 16
NW = NC * NS
B = 128
NCHUNK = E // B
KBUF = 79
KMAIN = 76
NBUF = 4
SBUF = 2
ROW_STRIDE = 624
ROW_CHUNKS = 5


def _agg_sc(y, edges, ew):
    mesh = plsc.VectorSubcoreMesh(core_axis_name="c", subcore_axis_name="s")

    @functools.partial(
        pl.kernel,
        mesh=mesh,
        compiler_params=pltpu.CompilerParams(use_tc_tiling_on_sc=False),
        out_type=jax.ShapeDtypeStruct((N, 2 * U), jnp.float32),
        scratch_types=[
            pltpu.VMEM((KBUF * B,), jnp.int32),
            pltpu.VMEM((KBUF * B,), jnp.int32),
            pltpu.VMEM((KBUF * B,), jnp.float32),
            pltpu.VMEM((NBUF, B, U), jnp.float32),
            pltpu.VMEM((SBUF, B, U), jnp.float32),
            pltpu.VMEM_SHARED((N, U), jnp.float32),
        ]
        + [pltpu.SemaphoreType.DMA] * (NBUF + SBUF),
    )
    def agg(y_hbm, edge_hbm, ew_hbm, out_hbm,
            src_v, dst_v, ew_v, rows_v, sc_v, acc, *sems):
        gsem = sems[:NBUF]
        ssem = sems[NBUF:]
        c = lax.axis_index("c")
        s = lax.axis_index("s")
        wid = c * NS + s
        k0 = (NCHUNK * wid) // NW
        nk = (NCHUNK * (wid + 1)) // NW - k0

        e0 = k0 * B
        pltpu.sync_copy(edge_hbm.at[0, pl.ds(e0, KBUF * B)], src_v)
        pltpu.sync_copy(edge_hbm.at[1, pl.ds(e0, KBUF * B)], dst_v)
        pltpu.sync_copy(ew_hbm.at[pl.ds(e0, KBUF * B)], ew_v)

        def issue_gather(k, b):
            return pltpu.async_copy(y_hbm.at[src_v.at[pl.ds(k * B, B)]],
                                    rows_v.at[b], gsem[b])

        def zrow(r, _):
            for j in range(U // 16):
                sc_v[0, r, pl.ds(j * 16, 16)] = jnp.zeros((16,), jnp.float32)
            return 0
        lax.fori_loop(0, B, zrow, 0)
        r0 = s * ROW_STRIDE
        for z in range(ROW_CHUNKS):
            pltpu.async_copy(sc_v.at[0], acc.at[pl.ds(r0 + z * B, B)], ssem[0])
        for b0 in range(NBUF):
            issue_gather(b0, b0)
        for z in range(ROW_CHUNKS):
            pltpu.make_async_copy(sc_v.at[0], acc.at[pl.ds(r0, B)],
                                  ssem[0]).wait()
        plsc.subcore_barrier()

        def body(k, b, sb):
            pltpu.make_async_copy(y_hbm.at[src_v.at[pl.ds(k * B, B)]],
                                  rows_v.at[b], gsem[b]).wait()

            @pl.when(k >= SBUF)
            def _():
                pltpu.make_async_copy(sc_v.at[sb],
                                      acc.at[dst_v.at[pl.ds((k - SBUF) * B, B)]],
                                      ssem[sb]).wait()

            @plsc.parallel_loop(0, B // 16, unroll=2)
            def scale(i):
                wv = ew_v[pl.ds(k * B + i * 16, 16)]
                for g in range(4):
                    es = [i * 16 + g * 4 + l for l in range(4)]
                    vals = [rows_v[b, e, pl.ds(j * 16, 16)]
                            for e in es for j in range(U // 16)]
                    outs = [vals[n] * wv[g * 4 + n // 4]
                            for n in range(len(vals))]
                    for n, e in enumerate(es):
                        for j in range(U // 16):
                            sc_v[sb, e, pl.ds(j * 16, 16)] = outs[n * 4 + j]

            pltpu.async_copy(sc_v.at[sb], acc.at[dst_v.at[pl.ds(k * B, B)]],
                             ssem[sb], add=True)

            @pl.when(k + NBUF < nk)
            def _():
                issue_gather(k + NBUF, b)

        def quad(k4, _):
            for b in range(NBUF):
                body(k4 * NBUF + b, b, b % SBUF)
            return 0
        lax.fori_loop(0, KMAIN // NBUF, quad, 0)

        body(jnp.int32(KMAIN), 0, 0)
        body(jnp.int32(KMAIN + 1), 1, 1)

        @pl.when(nk == KBUF)
        def _():
            body(jnp.int32(KMAIN + 2), 2, 0)

        for sb in range(SBUF):
            pltpu.make_async_copy(sc_v.at[sb], acc.at[dst_v.at[pl.ds(0, B)]],
                                  ssem[sb]).wait()
        plsc.subcore_barrier()

        bufs = [rows_v.at[0], rows_v.at[1], rows_v.at[2], rows_v.at[3],
                sc_v.at[0]]
        for z in range(ROW_CHUNKS):
            pltpu.async_copy(acc.at[pl.ds(r0 + z * B, B)], bufs[z], gsem[0])
        for z in range(ROW_CHUNKS):
            pltpu.make_async_copy(acc.at[pl.ds(r0 + z * B, B)], bufs[z],
                                  gsem[0]).wait()
        for z in range(ROW_CHUNKS):
            pltpu.async_copy(bufs[z],
                             out_hbm.at[pl.ds(r0 + z * B, B), pl.ds(c * U, U)],
                             ssem[0])
        for z in range(ROW_CHUNKS):
            pltpu.make_async_copy(bufs[z],
                                  out_hbm.at[pl.ds(r0 + z * B, B),
                                             pl.ds(c * U, U)],
                                  ssem[0]).wait()

    return agg(y, edges, ew)


_RB = 1000


def _mm1_body(x_ref, w_ref, o_ref):
    o_ref[...] = jnp.dot(x_ref[...], w_ref[...], preferred_element_type=jnp.float32)


def _mlp1_body(y_ref, p_ref, b1_ref, w2_ref, b2_ref, g1_ref, be1_ref,
               mm1_ref, mv1_ref, w3_ref, o_ref):
    p = p_ref[...]
    a = (1.0 + EPS_GIN) * y_ref[...] + p[:, :U] + p[:, U:] + b1_ref[...]
    h = jnp.maximum(a, 0.0)
    h = jnp.dot(h, w2_ref[...], preferred_element_type=jnp.float32) + b2_ref[...]
    scale = g1_ref[...] * lax.rsqrt(mv1_ref[...] + BN_EPS)
    h = (h - mm1_ref[...]) * scale + be1_ref[...]
    h = jnp.maximum(h, 0.0)
    o_ref[...] = jnp.dot(h, w3_ref[...], preferred_element_type=jnp.float32)


def _mlp2_body(y_ref, p_ref, b3_ref, w4_ref, b4_ref, g2_ref, be2_ref,
               mm2_ref, mv2_ref, o_ref):
    p = p_ref[...]
    a = (1.0 + EPS_GIN) * y_ref[...] + p[:, :U] + p[:, U:] + b3_ref[...]
    h = jnp.maximum(a, 0.0)
    o = jnp.dot(h, w4_ref[...], preferred_element_type=jnp.float32) + b4_ref[...]
    scale = g2_ref[...] * lax.rsqrt(mv2_ref[...] + BN_EPS)
    o_ref[...] = (o - mm2_ref[...]) * scale + be2_ref[...]


def _row(v):
    return v.reshape(1, -1)


def kernel(x, edge_index, edge_weight, W1, b1, W2, b2, g1, be1, mm1, mv1,
           W3, b3, W4, b4, g2, be2, mm2, mv2):
    edges = edge_index
    ew_p = edge_weight

    grid = (N // _RB,)

    y1 = pl.pallas_call(
        _mm1_body,
        grid=grid,
        in_specs=[pl.BlockSpec((_RB, D), lambda i: (i, 0)),
                  pl.BlockSpec((D, U), lambda i: (0, 0))],
        out_specs=pl.BlockSpec((_RB, U), lambda i: (i, 0)),
        out_shape=jax.ShapeDtypeStruct((N, U), jnp.float32),
    )(x, W1)

    p1 = _agg_sc(y1, edges, ew_p)

    y2 = pl.pallas_call(
        _mlp1_body,
        grid=grid,
        in_specs=[pl.BlockSpec((_RB, U), lambda i: (i, 0)),
                  pl.BlockSpec((_RB, 2 * U), lambda i: (i, 0)),
                  pl.BlockSpec((1, U), lambda i: (0, 0)),
                  pl.BlockSpec((U, U), lambda i: (0, 0)),
                  pl.BlockSpec((1, U), lambda i: (0, 0)),
                  pl.BlockSpec((1, U), lambda i: (0, 0)),
                  pl.BlockSpec((1, U), lambda i: (0, 0)),
                  pl.BlockSpec((1, U), lambda i: (0, 0)),
                  pl.BlockSpec((1, U), lambda i: (0, 0)),
                  pl.BlockSpec((U, U), lambda i: (0, 0))],
        out_specs=pl.BlockSpec((_RB, U), lambda i: (i, 0)),
        out_shape=jax.ShapeDtypeStruct((N, U), jnp.float32),
    )(y1, p1, _row(b1), W2, _row(b2), _row(g1), _row(be1), _row(mm1),
      _row(mv1), W3)

    p2 = _agg_sc(y2, edges, ew_p)

    out = pl.pallas_call(
        _mlp2_body,
        grid=grid,
        in_specs=[pl.BlockSpec((_RB, U), lambda i: (i, 0)),
                  pl.BlockSpec((_RB, 2 * U), lambda i: (i, 0)),
                  pl.BlockSpec((1, U), lambda i: (0, 0)),
                  pl.BlockSpec((U, C), lambda i: (0, 0)),
                  pl.BlockSpec((1, C), lambda i: (0, 0)),
                  pl.BlockSpec((1, C), lambda i: (0, 0)),
                  pl.BlockSpec((1, C), lambda i: (0, 0)),
                  pl.BlockSpec((1, C), lambda i: (0, 0)),
                  pl.BlockSpec((1, C), lambda i: (0, 0))],
        out_specs=pl.BlockSpec((_RB, C), lambda i: (i, 0)),
        out_shape=jax.ShapeDtypeStruct((N, C), jnp.float32),
    )(y2, p2, _row(b3), W4, _row(b4), _row(g2), _row(be2), _row(mm2),
      _row(mv2))

    return out


# speedup vs baseline: 1.0594x; 1.0324x over previous
import functools

import jax
import jax.numpy as jnp
from jax import lax
from jax.experimental import pallas as pl
from jax.experimental.pallas import tpu as pltpu
from jax.experimental.pallas import tpu_sc as plsc

N, E, D, U, C = 10000, 320000, 128, 64, 40
EPS_GIN = 0.5
BN_EPS = 1e-3

NC, NS = 2, 16
NW = NC * NS
B = 128
NCHUNK = E // B
KBUF = 79
KMAIN = 76
NBUF = 4
SBUF = 2
ROW_STRIDE = 624
ROW_CHUNKS = 5


def _agg_sc(y, edges, ew):
    mesh = plsc.VectorSubcoreMesh(core_axis_name="c", subcore_axis_name="s")

    @functools.partial(
        pl.kernel,
        mesh=mesh,
        compiler_params=pltpu.CompilerParams(use_tc_tiling_on_sc=False),
        out_type=jax.ShapeDtypeStruct((N, 2 * U), jnp.float32),
        scratch_types=[
            pltpu.VMEM((KBUF * B,), jnp.int32),
            pltpu.VMEM((KBUF * B,), jnp.int32),
            pltpu.VMEM((KBUF * B,), jnp.float32),
            pltpu.VMEM((NBUF, B, U), jnp.float32),
            pltpu.VMEM((SBUF, B, U), jnp.float32),
            pltpu.VMEM_SHARED((N, U), jnp.float32),
        ]
        + [pltpu.SemaphoreType.DMA] * (NBUF + SBUF),
    )
    def agg(y_hbm, edge_hbm, ew_hbm, out_hbm,
            src_v, dst_v, ew_v, rows_v, sc_v, acc, *sems):
        gsem = sems[:NBUF]
        ssem = sems[NBUF:]
        c = lax.axis_index("c")
        s = lax.axis_index("s")
        wid = c * NS + s
        k0 = (NCHUNK * wid) // NW
        nk = (NCHUNK * (wid + 1)) // NW - k0

        e0 = k0 * B
        pltpu.sync_copy(edge_hbm.at[0, pl.ds(e0, KBUF * B)], src_v)
        pltpu.sync_copy(edge_hbm.at[1, pl.ds(e0, KBUF * B)], dst_v)
        pltpu.sync_copy(ew_hbm.at[pl.ds(e0, KBUF * B)], ew_v)

        def issue_gather(k, b):
            return pltpu.async_copy(y_hbm.at[src_v.at[pl.ds(k * B, B)]],
                                    rows_v.at[b], gsem[b])

        def zrow(r, _):
            for j in range(U // 16):
                sc_v[0, r, pl.ds(j * 16, 16)] = jnp.zeros((16,), jnp.float32)
            return 0
        lax.fori_loop(0, B, zrow, 0)
        r0 = s * ROW_STRIDE
        for z in range(ROW_CHUNKS):
            pltpu.async_copy(sc_v.at[0], acc.at[pl.ds(r0 + z * B, B)], ssem[0])
        for b0 in range(NBUF):
            issue_gather(b0, b0)
        for z in range(ROW_CHUNKS):
            pltpu.make_async_copy(sc_v.at[0], acc.at[pl.ds(r0, B)],
                                  ssem[0]).wait()
        plsc.subcore_barrier()

        def body(k, b, sb):
            pltpu.make_async_copy(y_hbm.at[src_v.at[pl.ds(k * B, B)]],
                                  rows_v.at[b], gsem[b]).wait()

            @pl.when(k >= SBUF)
            def _():
                pltpu.make_async_copy(sc_v.at[sb],
                                      acc.at[dst_v.at[pl.ds((k - SBUF) * B, B)]],
                                      ssem[sb]).wait()

            @plsc.parallel_loop(0, B // 16, unroll=2)
            def scale(i):
                wv = ew_v[pl.ds(k * B + i * 16, 16)]
                for g in range(4):
                    es = [i * 16 + g * 4 + l for l in range(4)]
                    vals = [rows_v[b, e, pl.ds(j * 16, 16)]
                            for e in es for j in range(U // 16)]
                    outs = [vals[n] * wv[g * 4 + n // 4]
                            for n in range(len(vals))]
                    for n, e in enumerate(es):
                        for j in range(U // 16):
                            sc_v[sb, e, pl.ds(j * 16, 16)] = outs[n * 4 + j]

            pltpu.async_copy(sc_v.at[sb], acc.at[dst_v.at[pl.ds(k * B, B)]],
                             ssem[sb], add=True)

            @pl.when(k + NBUF < nk)
            def _():
                issue_gather(k + NBUF, b)

        def quad(k4, _):
            for b in range(NBUF):
                body(k4 * NBUF + b, b, b % SBUF)
            return 0
        lax.fori_loop(0, KMAIN // NBUF, quad, 0)

        body(jnp.int32(KMAIN), 0, 0)
        body(jnp.int32(KMAIN + 1), 1, 1)

        @pl.when(nk == KBUF)
        def _():
            body(jnp.int32(KMAIN + 2), 2, 0)

        for sb in range(SBUF):
            pltpu.make_async_copy(sc_v.at[sb], acc.at[dst_v.at[pl.ds(0, B)]],
                                  ssem[sb]).wait()
        plsc.subcore_barrier()

        for z in range(ROW_CHUNKS):
            pltpu.async_copy(acc.at[pl.ds(r0 + z * B, B)],
                             out_hbm.at[pl.ds(r0 + z * B, B), pl.ds(c * U, U)],
                             ssem[0])
        for z in range(ROW_CHUNKS):
            pltpu.make_async_copy(acc.at[pl.ds(r0 + z * B, B)],
                                  out_hbm.at[pl.ds(r0 + z * B, B),
                                             pl.ds(c * U, U)],
                                  ssem[0]).wait()

    return agg(y, edges, ew)


_RB = 2000


def _mm1_body(x_ref, w_ref, o_ref):
    o_ref[...] = jnp.dot(x_ref[...], w_ref[...], preferred_element_type=jnp.float32)


def _mlp1_body(y_ref, p_ref, b1_ref, w2_ref, b2_ref, g1_ref, be1_ref,
               mm1_ref, mv1_ref, w3_ref, o_ref):
    p = p_ref[...]
    a = (1.0 + EPS_GIN) * y_ref[...] + p[:, :U] + p[:, U:] + b1_ref[...]
    h = jnp.maximum(a, 0.0)
    h = jnp.dot(h, w2_ref[...], preferred_element_type=jnp.float32) + b2_ref[...]
    scale = g1_ref[...] * lax.rsqrt(mv1_ref[...] + BN_EPS)
    h = (h - mm1_ref[...]) * scale + be1_ref[...]
    h = jnp.maximum(h, 0.0)
    o_ref[...] = jnp.dot(h, w3_ref[...], preferred_element_type=jnp.float32)


def _mlp2_body(y_ref, p_ref, b3_ref, w4_ref, b4_ref, g2_ref, be2_ref,
               mm2_ref, mv2_ref, o_ref):
    p = p_ref[...]
    a = (1.0 + EPS_GIN) * y_ref[...] + p[:, :U] + p[:, U:] + b3_ref[...]
    h = jnp.maximum(a, 0.0)
    o = jnp.dot(h, w4_ref[...], preferred_element_type=jnp.float32) + b4_ref[...]
    scale = g2_ref[...] * lax.rsqrt(mv2_ref[...] + BN_EPS)
    o_ref[...] = (o - mm2_ref[...]) * scale + be2_ref[...]


def _row(v):
    return v.reshape(1, -1)


def kernel(x, edge_index, edge_weight, W1, b1, W2, b2, g1, be1, mm1, mv1,
           W3, b3, W4, b4, g2, be2, mm2, mv2):
    edges = edge_index
    ew_p = edge_weight

    grid = (N // _RB,)

    y1 = pl.pallas_call(
        _mm1_body,
        grid=grid,
        in_specs=[pl.BlockSpec((_RB, D), lambda i: (i, 0)),
                  pl.BlockSpec((D, U), lambda i: (0, 0))],
        out_specs=pl.BlockSpec((_RB, U), lambda i: (i, 0)),
        out_shape=jax.ShapeDtypeStruct((N, U), jnp.float32),
    )(x, W1)

    p1 = _agg_sc(y1, edges, ew_p)

    y2 = pl.pallas_call(
        _mlp1_body,
        grid=grid,
        in_specs=[pl.BlockSpec((_RB, U), lambda i: (i, 0)),
                  pl.BlockSpec((_RB, 2 * U), lambda i: (i, 0)),
                  pl.BlockSpec((1, U), lambda i: (0, 0)),
                  pl.BlockSpec((U, U), lambda i: (0, 0)),
                  pl.BlockSpec((1, U), lambda i: (0, 0)),
                  pl.BlockSpec((1, U), lambda i: (0, 0)),
                  pl.BlockSpec((1, U), lambda i: (0, 0)),
                  pl.BlockSpec((1, U), lambda i: (0, 0)),
                  pl.BlockSpec((1, U), lambda i: (0, 0)),
                  pl.BlockSpec((U, U), lambda i: (0, 0))],
        out_specs=pl.BlockSpec((_RB, U), lambda i: (i, 0)),
        out_shape=jax.ShapeDtypeStruct((N, U), jnp.float32),
    )(y1, p1, _row(b1), W2, _row(b2), _row(g1), _row(be1), _row(mm1),
      _row(mv1), W3)

    p2 = _agg_sc(y2, edges, ew_p)

    out = pl.pallas_call(
        _mlp2_body,
        grid=grid,
        in_specs=[pl.BlockSpec((_RB, U), lambda i: (i, 0)),
                  pl.BlockSpec((_RB, 2 * U), lambda i: (i, 0)),
                  pl.BlockSpec((1, U), lambda i: (0, 0)),
                  pl.BlockSpec((U, C), lambda i: (0, 0)),
                  pl.BlockSpec((1, C), lambda i: (0, 0)),
                  pl.BlockSpec((1, C), lambda i: (0, 0)),
                  pl.BlockSpec((1, C), lambda i: (0, 0)),
                  pl.BlockSpec((1, C), lambda i: (0, 0)),
                  pl.BlockSpec((1, C), lambda i: (0, 0))],
        out_specs=pl.BlockSpec((_RB, C), lambda i: (i, 0)),
        out_shape=jax.ShapeDtypeStruct((N, C), jnp.float32),
    )(y2, p2, _row(b3), W4, _row(b4), _row(g2), _row(be2), _row(mm2),
      _row(mv2))

    return out


# speedup vs baseline: 1.0935x; 1.0322x over previous
import functools

import jax
import jax.numpy as jnp
from jax import lax
from jax.experimental import pallas as pl
from jax.experimental.pallas import tpu as pltpu
from jax.experimental.pallas import tpu_sc as plsc

N, E, D, U, C = 10000, 320000, 128, 64, 40
EPS_GIN = 0.5
BN_EPS = 1e-3

NC, NS = 2, 16
NW = NC * NS
B = 128
NCHUNK = E // B
KBUF = 79
KMAIN = 76
NBUF = 4
SBUF = 2
ROW_STRIDE = 624
ROW_CHUNKS = 5


def _agg_sc(y, edges, ew):
    mesh = plsc.VectorSubcoreMesh(core_axis_name="c", subcore_axis_name="s")

    @functools.partial(
        pl.kernel,
        mesh=mesh,
        compiler_params=pltpu.CompilerParams(use_tc_tiling_on_sc=False),
        out_type=jax.ShapeDtypeStruct((N, 2 * U), jnp.float32),
        scratch_types=[
            pltpu.VMEM((KBUF * B,), jnp.int32),
            pltpu.VMEM((KBUF * B,), jnp.int32),
            pltpu.VMEM((KBUF * B,), jnp.float32),
            pltpu.VMEM((NBUF, B, U), jnp.float32),
            pltpu.VMEM((SBUF, B, U), jnp.float32),
            pltpu.VMEM_SHARED((N, U), jnp.float32),
        ]
        + [pltpu.SemaphoreType.DMA] * (NBUF + SBUF),
    )
    def agg(y_hbm, edge_hbm, ew_hbm, out_hbm,
            src_v, dst_v, ew_v, rows_v, sc_v, acc, *sems):
        gsem = sems[:NBUF]
        ssem = sems[NBUF:]
        c = lax.axis_index("c")
        s = lax.axis_index("s")
        wid = c * NS + s
        k0 = (NCHUNK * wid) // NW
        nk = (NCHUNK * (wid + 1)) // NW - k0

        e0 = k0 * B
        pltpu.sync_copy(edge_hbm.at[0, pl.ds(e0, KBUF * B)], src_v)
        pltpu.sync_copy(edge_hbm.at[1, pl.ds(e0, KBUF * B)], dst_v)
        pltpu.sync_copy(ew_hbm.at[pl.ds(e0, KBUF * B)], ew_v)

        def issue_gather(k, b):
            return pltpu.async_copy(y_hbm.at[src_v.at[pl.ds(k * B, B)]],
                                    rows_v.at[b], gsem[b])

        def zrow(r, _):
            for j in range(U // 16):
                sc_v[0, r, pl.ds(j * 16, 16)] = jnp.zeros((16,), jnp.float32)
            return 0
        lax.fori_loop(0, B, zrow, 0)
        r0 = s * ROW_STRIDE
        for z in range(ROW_CHUNKS):
            pltpu.async_copy(sc_v.at[0], acc.at[pl.ds(r0 + z * B, B)], ssem[0])
        for b0 in range(NBUF):
            issue_gather(b0, b0)
        for z in range(ROW_CHUNKS):
            pltpu.make_async_copy(sc_v.at[0], acc.at[pl.ds(r0, B)],
                                  ssem[0]).wait()
        plsc.subcore_barrier()

        def body(k, b, sb):
            pltpu.make_async_copy(y_hbm.at[src_v.at[pl.ds(k * B, B)]],
                                  rows_v.at[b], gsem[b]).wait()

            @pl.when(k >= SBUF)
            def _():
                pltpu.make_async_copy(sc_v.at[sb],
                                      acc.at[dst_v.at[pl.ds((k - SBUF) * B, B)]],
                                      ssem[sb]).wait()

            @plsc.parallel_loop(0, B // 16, unroll=2)
            def scale(i):
                wv = ew_v[pl.ds(k * B + i * 16, 16)]
                for g in range(4):
                    es = [i * 16 + g * 4 + l for l in range(4)]
                    vals = [rows_v[b, e, pl.ds(j * 16, 16)]
                            for e in es for j in range(U // 16)]
                    outs = [vals[n] * wv[g * 4 + n // 4]
                            for n in range(len(vals))]
                    for n, e in enumerate(es):
                        for j in range(U // 16):
                            sc_v[sb, e, pl.ds(j * 16, 16)] = outs[n * 4 + j]

            pltpu.async_copy(sc_v.at[sb], acc.at[dst_v.at[pl.ds(k * B, B)]],
                             ssem[sb], add=True)

            @pl.when(k + NBUF < nk)
            def _():
                issue_gather(k + NBUF, b)

        def quad(k4, _):
            for b in range(NBUF):
                body(k4 * NBUF + b, b, b % SBUF)
            return 0
        lax.fori_loop(0, KMAIN // NBUF, quad, 0)

        body(jnp.int32(KMAIN), 0, 0)
        body(jnp.int32(KMAIN + 1), 1, 1)

        @pl.when(nk == KBUF)
        def _():
            body(jnp.int32(KMAIN + 2), 2, 0)

        for sb in range(SBUF):
            pltpu.make_async_copy(sc_v.at[sb], acc.at[dst_v.at[pl.ds(0, B)]],
                                  ssem[sb]).wait()
        plsc.subcore_barrier()

        for z in range(ROW_CHUNKS):
            pltpu.async_copy(acc.at[pl.ds(r0 + z * B, B)],
                             out_hbm.at[pl.ds(r0 + z * B, B), pl.ds(c * U, U)],
                             ssem[0])
        for z in range(ROW_CHUNKS):
            pltpu.make_async_copy(acc.at[pl.ds(r0 + z * B, B)],
                                  out_hbm.at[pl.ds(r0 + z * B, B),
                                             pl.ds(c * U, U)],
                                  ssem[0]).wait()

    return agg(y, edges, ew)


_RB = 5000


def _mm1_body(x_ref, w_ref, o_ref):
    o_ref[...] = jnp.dot(x_ref[...], w_ref[...], preferred_element_type=jnp.float32)


def _mlp1_body(y_ref, p_ref, b1_ref, w2_ref, b2_ref, g1_ref, be1_ref,
               mm1_ref, mv1_ref, w3_ref, o_ref):
    p = p_ref[...]
    a = (1.0 + EPS_GIN) * y_ref[...] + p[:, :U] + p[:, U:] + b1_ref[...]
    h = jnp.maximum(a, 0.0)
    h = jnp.dot(h, w2_ref[...], preferred_element_type=jnp.float32) + b2_ref[...]
    scale = g1_ref[...] * lax.rsqrt(mv1_ref[...] + BN_EPS)
    h = (h - mm1_ref[...]) * scale + be1_ref[...]
    h = jnp.maximum(h, 0.0)
    o_ref[...] = jnp.dot(h, w3_ref[...], preferred_element_type=jnp.float32)


def _mlp2_body(y_ref, p_ref, b3_ref, w4_ref, b4_ref, g2_ref, be2_ref,
               mm2_ref, mv2_ref, o_ref):
    p = p_ref[...]
    a = (1.0 + EPS_GIN) * y_ref[...] + p[:, :U] + p[:, U:] + b3_ref[...]
    h = jnp.maximum(a, 0.0)
    o = jnp.dot(h, w4_ref[...], preferred_element_type=jnp.float32) + b4_ref[...]
    scale = g2_ref[...] * lax.rsqrt(mv2_ref[...] + BN_EPS)
    o_ref[...] = (o - mm2_ref[...]) * scale + be2_ref[...]


def _row(v):
    return v.reshape(1, -1)


def kernel(x, edge_index, edge_weight, W1, b1, W2, b2, g1, be1, mm1, mv1,
           W3, b3, W4, b4, g2, be2, mm2, mv2):
    edges = edge_index
    ew_p = edge_weight

    grid = (N // _RB,)

    y1 = pl.pallas_call(
        _mm1_body,
        grid=grid,
        in_specs=[pl.BlockSpec((_RB, D), lambda i: (i, 0)),
                  pl.BlockSpec((D, U), lambda i: (0, 0))],
        out_specs=pl.BlockSpec((_RB, U), lambda i: (i, 0)),
        out_shape=jax.ShapeDtypeStruct((N, U), jnp.float32),
    )(x, W1)

    p1 = _agg_sc(y1, edges, ew_p)

    y2 = pl.pallas_call(
        _mlp1_body,
        grid=grid,
        in_specs=[pl.BlockSpec((_RB, U), lambda i: (i, 0)),
                  pl.BlockSpec((_RB, 2 * U), lambda i: (i, 0)),
                  pl.BlockSpec((1, U), lambda i: (0, 0)),
                  pl.BlockSpec((U, U), lambda i: (0, 0)),
                  pl.BlockSpec((1, U), lambda i: (0, 0)),
                  pl.BlockSpec((1, U), lambda i: (0, 0)),
                  pl.BlockSpec((1, U), lambda i: (0, 0)),
                  pl.BlockSpec((1, U), lambda i: (0, 0)),
                  pl.BlockSpec((1, U), lambda i: (0, 0)),
                  pl.BlockSpec((U, U), lambda i: (0, 0))],
        out_specs=pl.BlockSpec((_RB, U), lambda i: (i, 0)),
        out_shape=jax.ShapeDtypeStruct((N, U), jnp.float32),
    )(y1, p1, _row(b1), W2, _row(b2), _row(g1), _row(be1), _row(mm1),
      _row(mv1), W3)

    p2 = _agg_sc(y2, edges, ew_p)

    out = pl.pallas_call(
        _mlp2_body,
        grid=grid,
        in_specs=[pl.BlockSpec((_RB, U), lambda i: (i, 0)),
                  pl.BlockSpec((_RB, 2 * U), lambda i: (i, 0)),
                  pl.BlockSpec((1, U), lambda i: (0, 0)),
                  pl.BlockSpec((U, C), lambda i: (0, 0)),
                  pl.BlockSpec((1, C), lambda i: (0, 0)),
                  pl.BlockSpec((1, C), lambda i: (0, 0)),
                  pl.BlockSpec((1, C), lambda i: (0, 0)),
                  pl.BlockSpec((1, C), lambda i: (0, 0)),
                  pl.BlockSpec((1, C), lambda i: (0, 0))],
        out_specs=pl.BlockSpec((_RB, C), lambda i: (i, 0)),
        out_shape=jax.ShapeDtypeStruct((N, C), jnp.float32),
    )(y2, p2, _row(b3), W4, _row(b4), _row(g2), _row(be2), _row(mm2),
      _row(mv2))

    return out
